# baseline (speedup 1.0000x reference)
import functools

import jax
import jax.numpy as jnp
from jax import lax
from jax.experimental import pallas as pl
from jax.experimental.pallas import tpu as pltpu
from jax.experimental.pallas import tpu_sc as plsc

BATCH = 16384
NF = 26
VOCAB = 100000
D = 32

NC = 2
NS = 16
L = 16
NW = NC * NS

R = BATCH * NF
RW = R // NW
JROWS = RW // 128
CROWS = 1024
NCHUNK = RW // CROWS
KB = CROWS // 128
ORPC = CROWS * D // 128
TROWS = NF * VOCAB // 4


def _emb_body(x_hbm, tab_hbm, out_hbm, xv, ridx, gbuf, obuf, gsem):
    wid = lax.axis_index("s") * NC + lax.axis_index("c")

    pltpu.sync_copy(x_hbm.at[pl.ds(wid * JROWS, JROWS), :], xv)

    def cvt(j, carry):
        for o in range(128 // L):
            pos = j * 128 + o * L + lax.iota(jnp.int32, L)
            v = xv[j, pl.ds(o * L, L)] + (pos % NF) * VOCAB
            xv[j, pl.ds(o * L, L)] = v
            ridx[j, pl.ds(o * L, L)] = v >> 2
        return carry

    lax.fori_loop(0, JROWS, cvt, 0)

    def fire(j, k):
        return pltpu.async_copy(tab_hbm.at[ridx.at[j]], gbuf.at[k % 2], gsem)

    def chunk(c, carry):
        handles = {0: fire(c * KB, 0)}
        for k in range(KB):
            if k + 1 < KB:
                handles[k + 1] = fire(c * KB + k + 1, k + 1)
            handles[k].wait()

            def extract(g, carry2, k=k):
                j = c * KB + k
                v = xv[j, pl.ds(g * L, L)]
                col0 = (v & 3) * D
                rowv = lax.iota(jnp.int32, L) + g * L
                o = k * 128 + g * L + lax.iota(jnp.int32, L)
                orow = o >> 2
                ocol0 = (o & 3) * D
                blk = gbuf.at[k % 2]
                for d in range(D):
                    vals = plsc.load_gather(blk, [rowv, col0 + d])
                    plsc.store_scatter(obuf, [orow, ocol0 + d], vals)
                return carry2

            lax.fori_loop(0, 128 // L, extract, 0)
        pltpu.sync_copy(obuf,
                        out_hbm.at[pl.ds(wid * (RW * D // 128) + c * ORPC,
                                         ORPC), :])
        return carry

    lax.fori_loop(0, NCHUNK, chunk, 0)


@jax.jit
def kernel(x, tables):
    x2d = x.reshape(R // 128, 128)
    tab = tables.reshape(TROWS, 128)
    mesh = plsc.VectorSubcoreMesh(core_axis_name="c", subcore_axis_name="s")
    out = pl.kernel(
        _emb_body,
        out_type=jax.ShapeDtypeStruct((R * D // 128, 128), jnp.float32),
        mesh=mesh,
        scratch_types=[
            pltpu.VMEM((JROWS, 128), jnp.int32),
            pltpu.VMEM((JROWS, 128), jnp.int32),
            pltpu.VMEM((2, 128, 128), jnp.float32),
            pltpu.VMEM((ORPC, 128), jnp.float32),
            pltpu.SemaphoreType.DMA,
        ],
        compiler_params=pltpu.CompilerParams(needs_layout_passes=False),
    )(x2d, tab)
    return out.reshape(BATCH, NF * D)


# speedup vs baseline: 1.3758x; 1.3758x over previous
import functools

import jax
import jax.numpy as jnp
from jax import lax
from jax.experimental import pallas as pl
from jax.experimental.pallas import tpu as pltpu
from jax.experimental.pallas import tpu_sc as plsc

BATCH = 16384
NF = 26
VOCAB = 100000
D = 32

NC = 2
NS = 16
L = 16
NW = NC * NS

R = BATCH * NF
RW = R // NW
JROWS = RW // 128
CROWS = 1664
NGRP = RW // CROWS
KJ = CROWS // 128


def _emb_body(x_hbm, tab_hbm, out_hbm, xidx, rows, semA, semB, semSA, semSB):
    wid = lax.axis_index("s") * NC + lax.axis_index("c")

    pltpu.sync_copy(x_hbm.at[pl.ds(wid * JROWS, JROWS), :], xidx)

    def cvt(j, carry):
        for o in range(128 // L):
            pos = j * 128 + o * L + lax.iota(jnp.int32, L)
            xidx[j, pl.ds(o * L, L)] = xidx[j, pl.ds(o * L, L)] + (pos % NF) * VOCAB
        return carry

    lax.fori_loop(0, JROWS, cvt, 0)

    gsem = [semA, semB]
    ssem = [semSA, semSB]

    def fire(g):
        buf, sem = g % 2, gsem[g % 2]
        return [
            pltpu.async_copy(tab_hbm.at[xidx.at[g * KJ + k]],
                             rows.at[buf, pl.ds(k * 128, 128), :], sem)
            for k in range(KJ)
        ]

    def fire_store(g):
        buf = g % 2
        return pltpu.async_copy(
            rows.at[buf],
            out_hbm.at[pl.ds(wid * RW + g * CROWS, CROWS), :], ssem[buf])

    gathers = {0: fire(0)}
    stores = {}
    for g in range(1, NGRP):
        if g >= 2:
            stores[g - 2].wait()
        gathers[g] = fire(g)
        for h in gathers[g - 1]:
            h.wait()
        stores[g - 1] = fire_store(g - 1)
    for h in gathers[NGRP - 1]:
        h.wait()
    stores[NGRP - 1] = fire_store(NGRP - 1)
    stores[NGRP - 2].wait()
    stores[NGRP - 1].wait()


@jax.jit
def kernel(x, tables):
    x2d = x.reshape(R // 128, 128)
    lin = (tables.transpose(0, 2, 1)
           .reshape(NF, D, VOCAB // 4, 4)
           .transpose(0, 2, 3, 1)
           .reshape(NF * VOCAB // 4, 4 * D))
    tab = lin.reshape(NF * VOCAB, D)
    mesh = plsc.VectorSubcoreMesh(core_axis_name="c", subcore_axis_name="s")
    out = pl.kernel(
        _emb_body,
        out_type=jax.ShapeDtypeStruct((R, D), jnp.float32),
        mesh=mesh,
        scratch_types=[
            pltpu.VMEM((JROWS, 128), jnp.int32),
            pltpu.VMEM((2, CROWS, D), jnp.float32),
            pltpu.SemaphoreType.DMA,
            pltpu.SemaphoreType.DMA,
            pltpu.SemaphoreType.DMA,
            pltpu.SemaphoreType.DMA,
        ],
        compiler_params=pltpu.CompilerParams(use_tc_tiling_on_sc=False),
    )(x2d, tab)
    return out.reshape(BATCH, NF * D)
